# Initial kernel scaffold; baseline (speedup 1.0000x reference)
import jax
import jax.numpy as jnp
from jax.experimental import pallas as pl


def kernel(inputs, targets):
    raise NotImplementedError("write your pallas kernel here")


# speedup vs baseline: 26.2083x; 26.2083x over previous
import functools

import jax
import jax.numpy as jnp
from jax import lax
from jax.experimental import pallas as pl
from jax.experimental.pallas import tpu as pltpu
from jax.experimental.pallas import tpu_sc as plsc

M = 16 * 512 * 512
B = 2048
NW = 32
W = M // NW
CHUNK = 8192
NCH = W // CHUNK
ROWS, COLS = 512, 8192
RBLK = 64
NLANES = 16


def _stats_body(l_ref, t_ref, max_ref, p_ref):
    i = pl.program_id(0)
    l = l_ref[...]
    t = t_ref[...].astype(jnp.float32)
    e = 1.0 - l * (2.0 * t - 1.0)
    bmax = jnp.max(e)
    bsum = jnp.sum(t)

    @pl.when(i == 0)
    def _():
        max_ref[0, 0] = bmax
        p_ref[0, 0] = bsum

    @pl.when(i > 0)
    def _():
        max_ref[0, 0] = jnp.maximum(max_ref[0, 0], bmax)
        p_ref[0, 0] = p_ref[0, 0] + bsum


def _stats(logits2d, targets2d):
    return pl.pallas_call(
        _stats_body,
        grid=(ROWS // RBLK,),
        in_specs=[
            pl.BlockSpec((RBLK, COLS), lambda i: (i, 0)),
            pl.BlockSpec((RBLK, COLS), lambda i: (i, 0)),
        ],
        out_specs=[
            pl.BlockSpec(memory_space=pltpu.SMEM),
            pl.BlockSpec(memory_space=pltpu.SMEM),
        ],
        out_shape=[
            jax.ShapeDtypeStruct((1, 1), jnp.float32),
            jax.ShapeDtypeStruct((1, 1), jnp.float32),
        ],
    )(logits2d, targets2d)


_mesh = plsc.VectorSubcoreMesh(core_axis_name="c", subcore_axis_name="s")


@functools.partial(
    pl.kernel,
    mesh=_mesh,
    compiler_params=pltpu.CompilerParams(needs_layout_passes=False),
    out_type=jax.ShapeDtypeStruct((NW, 2 * B), jnp.float32),
    scratch_types=[
        pltpu.VMEM((CHUNK,), jnp.float32),
        pltpu.VMEM((CHUNK,), jnp.int32),
        pltpu.VMEM((NLANES,), jnp.float32),
        pltpu.VMEM((NLANES * 2 * B,), jnp.float32),
        pltpu.VMEM((2 * B,), jnp.float32),
    ],
)
def _sc_hist(l_hbm, t_hbm, s_hbm, out_hbm, lbuf, tbuf, sbuf, hist, rhist):
    wid = lax.axis_index("s") * 2 + lax.axis_index("c")
    base = wid * W

    pltpu.sync_copy(s_hbm, sbuf)
    tmax = jnp.maximum(sbuf[...], 1e-30)
    invd = jnp.float32(B) / tmax
    lane_base = lax.broadcasted_iota(jnp.int32, (NLANES,), 0) * (2 * B)
    ones = jnp.full((NLANES,), 1.0, jnp.float32)

    def zbody(i, _):
        hist[pl.ds(i * NLANES, NLANES)] = jnp.zeros((NLANES,), jnp.float32)
        return ()

    lax.fori_loop(0, NLANES * 2 * B // NLANES, zbody, ())

    def chunk_body(c, _):
        off = base + c * CHUNK
        pltpu.sync_copy(l_hbm.at[pl.ds(off, CHUNK)], lbuf)
        pltpu.sync_copy(t_hbm.at[pl.ds(off, CHUNK)], tbuf)

        def vbody(i, _):
            lv = lbuf[pl.ds(i * NLANES, NLANES)]
            tv = tbuf[pl.ds(i * NLANES, NLANES)]
            tf = tv.astype(jnp.float32)
            e = 1.0 - lv * (2.0 * tf - 1.0)
            bi = jnp.minimum((e * invd).astype(jnp.int32), B - 1)
            idx = lane_base + tv * B + bi
            plsc.addupdate_scatter(hist, [idx], ones, mask=e > 0.0)
            return ()

        lax.fori_loop(0, CHUNK // NLANES, vbody, ())
        return ()

    lax.fori_loop(0, NCH, chunk_body, ())

    def rbody(j, _):
        acc = jnp.zeros((NLANES,), jnp.float32)
        for ln in range(NLANES):
            acc = acc + hist[pl.ds(ln * 2 * B + j * NLANES, NLANES)]
        rhist[pl.ds(j * NLANES, NLANES)] = acc
        return ()

    lax.fori_loop(0, 2 * B // NLANES, rbody, ())
    pltpu.sync_copy(rhist, out_hbm.at[wid])


def _finish_body(h_ref, tmax_ref, p_ref, out_ref):
    h = h_ref[...]
    hp = jnp.sum(h[:, :B], axis=0, keepdims=True)
    hn = jnp.sum(h[:, B:], axis=0, keepdims=True)
    r = lax.broadcasted_iota(jnp.int32, (B, 1), 0)
    c = lax.broadcasted_iota(jnp.int32, (1, B), 1)
    upper = (r >= c).astype(jnp.float32)
    sp = jnp.dot(hp, upper, precision=lax.Precision.HIGHEST,
                 preferred_element_type=jnp.float32)
    sn = jnp.dot(hn, upper, precision=lax.Precision.HIGHEST,
                 preferred_element_type=jnp.float32)
    p_tot = p_ref[0, 0]
    tmax = jnp.maximum(tmax_ref[0, 0], 1e-30)
    jac = 1.0 - (p_tot - sp) / jnp.maximum(p_tot + sn, 1.0)
    delta = tmax / jnp.float32(B)
    total = delta * (jnp.sum(jac) - 0.5 * jac[0, 0])
    out_ref[0, 0] = jnp.where(p_tot > 0.0, total, 0.0)


def _finish(hists, tmax, p_tot):
    return pl.pallas_call(
        _finish_body,
        in_specs=[
            pl.BlockSpec((NW, 2 * B), lambda: (0, 0)),
            pl.BlockSpec(memory_space=pltpu.SMEM),
            pl.BlockSpec(memory_space=pltpu.SMEM),
        ],
        out_specs=pl.BlockSpec(memory_space=pltpu.SMEM),
        out_shape=jax.ShapeDtypeStruct((1, 1), jnp.float32),
    )(hists, tmax, p_tot)


def kernel(inputs, targets):
    logits2d = inputs.reshape(ROWS, COLS)
    targets2d = targets.reshape(ROWS, COLS)
    tmax, p_tot = _stats(logits2d, targets2d)
    stats16 = jnp.broadcast_to(tmax.reshape(1), (NLANES,))
    hists = _sc_hist(logits2d.reshape(-1), targets2d.reshape(-1), stats16)
    loss = _finish(hists, tmax, p_tot)
    return loss.reshape(())


# speedup vs baseline: 36.3708x; 1.3878x over previous
import functools

import jax
import jax.numpy as jnp
from jax import lax
from jax.experimental import pallas as pl
from jax.experimental.pallas import tpu as pltpu
from jax.experimental.pallas import tpu_sc as plsc

M = 16 * 512 * 512
B = 2048
NW = 32
W = M // NW
CHUNK = 16384
NCH = W // CHUNK
ROWS, COLS = 512, 8192
RBLK = 64
NLANES = 16
UNROLL = 8


def _stats_body(l_ref, t_ref, e_ref, max_ref, p_ref):
    i = pl.program_id(0)
    l = l_ref[...]
    t = t_ref[...]
    tf = t.astype(jnp.float32)
    e = 1.0 - l * (2.0 * tf - 1.0)
    ebits = lax.bitcast_convert_type(e, jnp.int32)
    epacked = lax.bitcast_convert_type(
        jnp.bitwise_or(jnp.bitwise_and(ebits, jnp.int32(-2)), t), jnp.float32)
    e_ref[...] = epacked.reshape(RBLK * COLS)
    bmax = jnp.max(epacked)
    bsum = jnp.sum(tf)

    @pl.when(i == 0)
    def _():
        max_ref[0, 0] = bmax
        p_ref[0, 0] = bsum

    @pl.when(i > 0)
    def _():
        max_ref[0, 0] = jnp.maximum(max_ref[0, 0], bmax)
        p_ref[0, 0] = p_ref[0, 0] + bsum


def _stats(logits2d, targets2d):
    return pl.pallas_call(
        _stats_body,
        grid=(ROWS // RBLK,),
        in_specs=[
            pl.BlockSpec((RBLK, COLS), lambda i: (i, 0)),
            pl.BlockSpec((RBLK, COLS), lambda i: (i, 0)),
        ],
        out_specs=[
            pl.BlockSpec((RBLK * COLS,), lambda i: (i,)),
            pl.BlockSpec(memory_space=pltpu.SMEM),
            pl.BlockSpec(memory_space=pltpu.SMEM),
        ],
        out_shape=[
            jax.ShapeDtypeStruct((M,), jnp.float32),
            jax.ShapeDtypeStruct((1, 1), jnp.float32),
            jax.ShapeDtypeStruct((1, 1), jnp.float32),
        ],
    )(logits2d, targets2d)


_mesh = plsc.VectorSubcoreMesh(core_axis_name="c", subcore_axis_name="s")


@functools.partial(
    pl.kernel,
    mesh=_mesh,
    compiler_params=pltpu.CompilerParams(needs_layout_passes=False),
    out_type=jax.ShapeDtypeStruct((NW, 2 * B), jnp.float32),
    scratch_types=[
        pltpu.VMEM((CHUNK,), jnp.float32),
        pltpu.VMEM((CHUNK,), jnp.float32),
        pltpu.VMEM((NLANES,), jnp.float32),
        pltpu.VMEM((NLANES * 2 * B,), jnp.float32),
        pltpu.VMEM((2 * B,), jnp.float32),
        pltpu.SemaphoreType.DMA,
        pltpu.SemaphoreType.DMA,
    ],
)
def _sc_hist(e_hbm, s_hbm, out_hbm, bufa, bufb, sbuf, hist, rhist, sema, semb):
    wid = lax.axis_index("s") * 2 + lax.axis_index("c")
    base = wid * W

    pltpu.sync_copy(s_hbm, sbuf)
    tmax = jnp.maximum(sbuf[...], 1e-30)
    invd = jnp.float32(B) / tmax
    lane_base = lax.broadcasted_iota(jnp.int32, (NLANES,), 0) * (2 * B)
    ones = jnp.full((NLANES,), 1.0, jnp.float32)

    def zbody(i, _):
        hist[pl.ds(i * NLANES, NLANES)] = jnp.zeros((NLANES,), jnp.float32)
        return ()

    lax.fori_loop(0, NLANES * 2 * B // NLANES, zbody, ())

    def process(buf):
        def vbody(i, _):
            for u in range(UNROLL):
                v = buf[pl.ds((i * UNROLL + u) * NLANES, NLANES)]
                ti = jnp.bitwise_and(plsc.bitcast(v, jnp.int32), 1)
                bi = jnp.minimum((v * invd).astype(jnp.int32), B - 1)
                idx = lane_base + ti * B + bi
                plsc.addupdate_scatter(hist, [idx], ones, mask=v > 0.0)
            return ()

        lax.fori_loop(0, CHUNK // NLANES // UNROLL, vbody, ())

    def start(chunk, buf, sem):
        pltpu.async_copy(e_hbm.at[pl.ds(base + chunk * CHUNK, CHUNK)], buf, sem)

    def wait(chunk, buf, sem):
        pltpu.make_async_copy(
            e_hbm.at[pl.ds(base + chunk * CHUNK, CHUNK)], buf, sem).wait()

    start(0, bufa, sema)

    def chunk_body(h, _):
        ca = 2 * h
        start(ca + 1, bufb, semb)
        wait(ca, bufa, sema)
        process(bufa)
        start(jnp.minimum(ca + 2, NCH - 1), bufa, sema)
        wait(ca + 1, bufb, semb)
        process(bufb)
        return ()

    lax.fori_loop(0, NCH // 2, chunk_body, ())
    wait(NCH - 1, bufa, sema)

    def rbody(j, _):
        acc = jnp.zeros((NLANES,), jnp.float32)
        for ln in range(NLANES):
            acc = acc + hist[pl.ds(ln * 2 * B + j * NLANES, NLANES)]
        rhist[pl.ds(j * NLANES, NLANES)] = acc
        return ()

    lax.fori_loop(0, 2 * B // NLANES, rbody, ())
    pltpu.sync_copy(rhist, out_hbm.at[wid])


def _finish_body(h_ref, tmax_ref, p_ref, out_ref):
    h = h_ref[...]
    hp = jnp.sum(h[:, :B], axis=0, keepdims=True)
    hn = jnp.sum(h[:, B:], axis=0, keepdims=True)
    r = lax.broadcasted_iota(jnp.int32, (B, 1), 0)
    c = lax.broadcasted_iota(jnp.int32, (1, B), 1)
    upper = (r >= c).astype(jnp.float32)
    sp = jnp.dot(hp, upper, precision=lax.Precision.HIGHEST,
                 preferred_element_type=jnp.float32)
    sn = jnp.dot(hn, upper, precision=lax.Precision.HIGHEST,
                 preferred_element_type=jnp.float32)
    p_tot = p_ref[0, 0]
    tmax = jnp.maximum(tmax_ref[0, 0], 1e-30)
    jac = 1.0 - (p_tot - sp) / jnp.maximum(p_tot + sn, 1.0)
    delta = tmax / jnp.float32(B)
    total = delta * (jnp.sum(jac) - 0.5 * jac[0, 0])
    out_ref[0, 0] = jnp.where(p_tot > 0.0, total, 0.0)


def _finish(hists, tmax, p_tot):
    return pl.pallas_call(
        _finish_body,
        in_specs=[
            pl.BlockSpec((NW, 2 * B), lambda: (0, 0)),
            pl.BlockSpec(memory_space=pltpu.SMEM),
            pl.BlockSpec(memory_space=pltpu.SMEM),
        ],
        out_specs=pl.BlockSpec(memory_space=pltpu.SMEM),
        out_shape=jax.ShapeDtypeStruct((1, 1), jnp.float32),
    )(hists, tmax, p_tot)


def kernel(inputs, targets):
    logits2d = inputs.reshape(ROWS, COLS)
    targets2d = targets.reshape(ROWS, COLS)
    epacked, tmax, p_tot = _stats(logits2d, targets2d)
    stats16 = jnp.broadcast_to(tmax.reshape(1), (NLANES,))
    hists = _sc_hist(epacked, stats16)
    loss = _finish(hists, tmax, p_tot)
    return loss.reshape(())


# speedup vs baseline: 58.0057x; 1.5948x over previous
import functools

import jax
import jax.numpy as jnp
from jax import lax
from jax.experimental import pallas as pl
from jax.experimental.pallas import tpu as pltpu
from jax.experimental.pallas import tpu_sc as plsc

M = 16 * 512 * 512
B = 2048
NW = 32
W = M // NW
CHUNK = 16384
NCH = W // CHUNK
ROWS, COLS = 512, 8192
RBLK = 64
NLANES = 16
UNROLL = 8


def _stats_body(l_ref, t_ref, e_ref, max_ref, p_ref):
    i = pl.program_id(0)
    l = l_ref[...]
    t = t_ref[...]
    tf = t.astype(jnp.float32)
    e = 1.0 - l * (2.0 * tf - 1.0)
    ebits = lax.bitcast_convert_type(e, jnp.int32)
    epacked = lax.bitcast_convert_type(
        jnp.bitwise_or(jnp.bitwise_and(ebits, jnp.int32(-2)), t), jnp.float32)
    e_ref[...] = epacked.reshape(RBLK * COLS)
    bmax = jnp.max(epacked)
    bsum = jnp.sum(tf)

    @pl.when(i == 0)
    def _():
        max_ref[0, 0] = bmax
        p_ref[0, 0] = bsum

    @pl.when(i > 0)
    def _():
        max_ref[0, 0] = jnp.maximum(max_ref[0, 0], bmax)
        p_ref[0, 0] = p_ref[0, 0] + bsum


def _stats(logits2d, targets2d):
    return pl.pallas_call(
        _stats_body,
        grid=(ROWS // RBLK,),
        in_specs=[
            pl.BlockSpec((RBLK, COLS), lambda i: (i, 0)),
            pl.BlockSpec((RBLK, COLS), lambda i: (i, 0)),
        ],
        out_specs=[
            pl.BlockSpec((RBLK * COLS,), lambda i: (i,)),
            pl.BlockSpec(memory_space=pltpu.SMEM),
            pl.BlockSpec(memory_space=pltpu.SMEM),
        ],
        out_shape=[
            jax.ShapeDtypeStruct((M,), jnp.float32),
            jax.ShapeDtypeStruct((1, 1), jnp.float32),
            jax.ShapeDtypeStruct((1, 1), jnp.float32),
        ],
    )(logits2d, targets2d)


_mesh = plsc.VectorSubcoreMesh(core_axis_name="c", subcore_axis_name="s")


@functools.partial(
    pl.kernel,
    mesh=_mesh,
    compiler_params=pltpu.CompilerParams(needs_layout_passes=False),
    out_type=jax.ShapeDtypeStruct((NW, 2 * B), jnp.float32),
    scratch_types=[
        pltpu.VMEM((CHUNK,), jnp.float32),
        pltpu.VMEM((CHUNK,), jnp.float32),
        pltpu.VMEM((NLANES,), jnp.float32),
        pltpu.VMEM((NLANES * 2 * B,), jnp.float32),
        pltpu.VMEM((2 * B,), jnp.float32),
        pltpu.SemaphoreType.DMA,
        pltpu.SemaphoreType.DMA,
    ],
)
def _sc_hist(e_hbm, s_hbm, out_hbm, bufa, bufb, sbuf, hist, rhist, sema, semb):
    wid = lax.axis_index("s") * 2 + lax.axis_index("c")
    base = wid * W

    pltpu.sync_copy(s_hbm, sbuf)
    tmax = jnp.maximum(sbuf[...], 1e-30)
    invd = jnp.float32(B) / tmax
    lane_base = lax.broadcasted_iota(jnp.int32, (NLANES,), 0) * (2 * B)
    ones = jnp.full((NLANES,), 1.0, jnp.float32)

    def zbody(i, _):
        hist[pl.ds(i * NLANES, NLANES)] = jnp.zeros((NLANES,), jnp.float32)
        return ()

    lax.fori_loop(0, NLANES * 2 * B // NLANES, zbody, ())

    def process(buf):
        @plsc.parallel_loop(0, CHUNK // NLANES, unroll=UNROLL)
        def _(i):
            v = buf[pl.ds(i * NLANES, NLANES)]
            ti = jnp.bitwise_and(plsc.bitcast(v, jnp.int32), 1)
            bi = jnp.minimum((v * invd).astype(jnp.int32), B - 1)
            idx = lane_base + ti * B + bi
            plsc.addupdate_scatter(hist, [idx], ones, mask=v > 0.0)

    def start(chunk, buf, sem):
        pltpu.async_copy(e_hbm.at[pl.ds(base + chunk * CHUNK, CHUNK)], buf, sem)

    def wait(chunk, buf, sem):
        pltpu.make_async_copy(
            e_hbm.at[pl.ds(base + chunk * CHUNK, CHUNK)], buf, sem).wait()

    start(0, bufa, sema)

    def chunk_body(h, _):
        ca = 2 * h
        start(ca + 1, bufb, semb)
        wait(ca, bufa, sema)
        process(bufa)
        start(jnp.minimum(ca + 2, NCH - 1), bufa, sema)
        wait(ca + 1, bufb, semb)
        process(bufb)
        return ()

    lax.fori_loop(0, NCH // 2, chunk_body, ())
    wait(NCH - 1, bufa, sema)

    def rbody(j, _):
        acc = jnp.zeros((NLANES,), jnp.float32)
        for ln in range(NLANES):
            acc = acc + hist[pl.ds(ln * 2 * B + j * NLANES, NLANES)]
        rhist[pl.ds(j * NLANES, NLANES)] = acc
        return ()

    lax.fori_loop(0, 2 * B // NLANES, rbody, ())
    pltpu.sync_copy(rhist, out_hbm.at[wid])


def _finish_body(h_ref, tmax_ref, p_ref, out_ref):
    h = h_ref[...]
    hp = jnp.sum(h[:, :B], axis=0, keepdims=True)
    hn = jnp.sum(h[:, B:], axis=0, keepdims=True)
    r = lax.broadcasted_iota(jnp.int32, (B, 1), 0)
    c = lax.broadcasted_iota(jnp.int32, (1, B), 1)
    upper = (r >= c).astype(jnp.float32)
    sp = jnp.dot(hp, upper, precision=lax.Precision.HIGHEST,
                 preferred_element_type=jnp.float32)
    sn = jnp.dot(hn, upper, precision=lax.Precision.HIGHEST,
                 preferred_element_type=jnp.float32)
    p_tot = p_ref[0, 0]
    tmax = jnp.maximum(tmax_ref[0, 0], 1e-30)
    jac = 1.0 - (p_tot - sp) / jnp.maximum(p_tot + sn, 1.0)
    delta = tmax / jnp.float32(B)
    total = delta * (jnp.sum(jac) - 0.5 * jac[0, 0])
    out_ref[0, 0] = jnp.where(p_tot > 0.0, total, 0.0)


def _finish(hists, tmax, p_tot):
    return pl.pallas_call(
        _finish_body,
        in_specs=[
            pl.BlockSpec((NW, 2 * B), lambda: (0, 0)),
            pl.BlockSpec(memory_space=pltpu.SMEM),
            pl.BlockSpec(memory_space=pltpu.SMEM),
        ],
        out_specs=pl.BlockSpec(memory_space=pltpu.SMEM),
        out_shape=jax.ShapeDtypeStruct((1, 1), jnp.float32),
    )(hists, tmax, p_tot)


def kernel(inputs, targets):
    logits2d = inputs.reshape(ROWS, COLS)
    targets2d = targets.reshape(ROWS, COLS)
    epacked, tmax, p_tot = _stats(logits2d, targets2d)
    stats16 = jnp.broadcast_to(tmax.reshape(1), (NLANES,))
    hists = _sc_hist(epacked, stats16)
    loss = _finish(hists, tmax, p_tot)
    return loss.reshape(())


# speedup vs baseline: 59.3148x; 1.0226x over previous
import functools

import jax
import jax.numpy as jnp
from jax import lax
from jax.experimental import pallas as pl
from jax.experimental.pallas import tpu as pltpu
from jax.experimental.pallas import tpu_sc as plsc

M = 16 * 512 * 512
B = 2048
NW = 32
W = M // NW
CHUNK = 16384
NCH = W // CHUNK
ROWS, COLS = 512, 8192
RBLK = 64
NLANES = 16
UNROLL = 8


def _stats_body(l_ref, t_ref, e_ref, max_ref, p_ref):
    i = pl.program_id(0)
    l = l_ref[...]
    t = t_ref[...]
    tf = t.astype(jnp.float32)
    e = 1.0 - l * (2.0 * tf - 1.0)
    ebits = lax.bitcast_convert_type(e, jnp.int32)
    epacked = lax.bitcast_convert_type(
        jnp.bitwise_or(jnp.bitwise_and(ebits, jnp.int32(-2)), t), jnp.float32)
    e_ref[...] = epacked.reshape(RBLK * COLS)
    bmax = jnp.max(epacked)
    bsum = jnp.sum(tf)

    @pl.when(i == 0)
    def _():
        max_ref[0, 0] = bmax
        p_ref[0, 0] = bsum

    @pl.when(i > 0)
    def _():
        max_ref[0, 0] = jnp.maximum(max_ref[0, 0], bmax)
        p_ref[0, 0] = p_ref[0, 0] + bsum


def _stats(logits2d, targets2d):
    return pl.pallas_call(
        _stats_body,
        grid=(ROWS // RBLK,),
        in_specs=[
            pl.BlockSpec((RBLK, COLS), lambda i: (i, 0)),
            pl.BlockSpec((RBLK, COLS), lambda i: (i, 0)),
        ],
        out_specs=[
            pl.BlockSpec((RBLK * COLS,), lambda i: (i,)),
            pl.BlockSpec(memory_space=pltpu.SMEM),
            pl.BlockSpec(memory_space=pltpu.SMEM),
        ],
        out_shape=[
            jax.ShapeDtypeStruct((M,), jnp.float32),
            jax.ShapeDtypeStruct((1, 1), jnp.float32),
            jax.ShapeDtypeStruct((1, 1), jnp.float32),
        ],
    )(logits2d, targets2d)


_mesh = plsc.VectorSubcoreMesh(core_axis_name="c", subcore_axis_name="s")


@functools.partial(
    pl.kernel,
    mesh=_mesh,
    compiler_params=pltpu.CompilerParams(needs_layout_passes=False),
    out_type=jax.ShapeDtypeStruct((NW, 2 * B), jnp.float32),
    scratch_types=[
        pltpu.VMEM((CHUNK,), jnp.float32),
        pltpu.VMEM((CHUNK,), jnp.float32),
        pltpu.VMEM((NLANES,), jnp.float32),
        pltpu.VMEM((NLANES * (2 * B + NLANES),), jnp.float32),
        pltpu.VMEM((2 * B,), jnp.float32),
        pltpu.SemaphoreType.DMA,
        pltpu.SemaphoreType.DMA,
    ],
)
def _sc_hist(e_hbm, s_hbm, out_hbm, bufa, bufb, sbuf, hist, rhist, sema, semb):
    wid = lax.axis_index("s") * 2 + lax.axis_index("c")
    base = wid * W

    pltpu.sync_copy(s_hbm, sbuf)
    tmax = jnp.maximum(sbuf[...], 1e-30)
    invd = jnp.float32(B) / tmax
    lane = lax.broadcasted_iota(jnp.int32, (NLANES,), 0)
    lane_base = lane * (2 * B + NLANES) + lane
    ones = jnp.full((NLANES,), 1.0, jnp.float32)

    def zbody(i, _):
        hist[pl.ds(i * NLANES, NLANES)] = jnp.zeros((NLANES,), jnp.float32)
        return ()

    lax.fori_loop(0, NLANES * (2 * B + NLANES) // NLANES, zbody, ())

    def process(buf):
        @plsc.parallel_loop(0, CHUNK // NLANES, unroll=UNROLL)
        def _(i):
            v = buf[pl.ds(i * NLANES, NLANES)]
            ti = jnp.bitwise_and(plsc.bitcast(v, jnp.int32), 1)
            bi = jnp.minimum((v * invd).astype(jnp.int32), B - 1)
            idx = lane_base + ti * B + bi
            plsc.addupdate_scatter(hist, [idx], ones, mask=v > 0.0)

    def start(chunk, buf, sem):
        pltpu.async_copy(e_hbm.at[pl.ds(base + chunk * CHUNK, CHUNK)], buf, sem)

    def wait(chunk, buf, sem):
        pltpu.make_async_copy(
            e_hbm.at[pl.ds(base + chunk * CHUNK, CHUNK)], buf, sem).wait()

    start(0, bufa, sema)

    def chunk_body(h, _):
        ca = 2 * h
        start(ca + 1, bufb, semb)
        wait(ca, bufa, sema)
        process(bufa)
        start(jnp.minimum(ca + 2, NCH - 1), bufa, sema)
        wait(ca + 1, bufb, semb)
        process(bufb)
        return ()

    lax.fori_loop(0, NCH // 2, chunk_body, ())
    wait(NCH - 1, bufa, sema)

    def rbody(j, _):
        acc = jnp.zeros((NLANES,), jnp.float32)
        for ln in range(NLANES):
            acc = acc + hist[pl.ds(ln * (2 * B + NLANES) + ln + j * NLANES,
                                   NLANES)]
        rhist[pl.ds(j * NLANES, NLANES)] = acc
        return ()

    lax.fori_loop(0, 2 * B // NLANES, rbody, ())
    pltpu.sync_copy(rhist, out_hbm.at[wid])


def _finish_body(h_ref, tmax_ref, p_ref, out_ref):
    h = h_ref[...]
    hp = jnp.sum(h[:, :B], axis=0, keepdims=True)
    hn = jnp.sum(h[:, B:], axis=0, keepdims=True)
    r = lax.broadcasted_iota(jnp.int32, (B, 1), 0)
    c = lax.broadcasted_iota(jnp.int32, (1, B), 1)
    upper = (r >= c).astype(jnp.float32)
    sp = jnp.dot(hp, upper, precision=lax.Precision.HIGHEST,
                 preferred_element_type=jnp.float32)
    sn = jnp.dot(hn, upper, precision=lax.Precision.HIGHEST,
                 preferred_element_type=jnp.float32)
    p_tot = p_ref[0, 0]
    tmax = jnp.maximum(tmax_ref[0, 0], 1e-30)
    jac = 1.0 - (p_tot - sp) / jnp.maximum(p_tot + sn, 1.0)
    delta = tmax / jnp.float32(B)
    total = delta * (jnp.sum(jac) - 0.5 * jac[0, 0])
    out_ref[0, 0] = jnp.where(p_tot > 0.0, total, 0.0)


def _finish(hists, tmax, p_tot):
    return pl.pallas_call(
        _finish_body,
        in_specs=[
            pl.BlockSpec((NW, 2 * B), lambda: (0, 0)),
            pl.BlockSpec(memory_space=pltpu.SMEM),
            pl.BlockSpec(memory_space=pltpu.SMEM),
        ],
        out_specs=pl.BlockSpec(memory_space=pltpu.SMEM),
        out_shape=jax.ShapeDtypeStruct((1, 1), jnp.float32),
    )(hists, tmax, p_tot)


def kernel(inputs, targets):
    logits2d = inputs.reshape(ROWS, COLS)
    targets2d = targets.reshape(ROWS, COLS)
    epacked, tmax, p_tot = _stats(logits2d, targets2d)
    stats16 = jnp.broadcast_to(tmax.reshape(1), (NLANES,))
    hists = _sc_hist(epacked, stats16)
    loss = _finish(hists, tmax, p_tot)
    return loss.reshape(())


# speedup vs baseline: 87.7523x; 1.4794x over previous
import functools

import jax
import jax.numpy as jnp
from jax import lax
from jax.experimental import pallas as pl
from jax.experimental.pallas import tpu as pltpu
from jax.experimental.pallas import tpu_sc as plsc

M = 16 * 512 * 512
B = 2048
NW = 32
W = M // NW
CHUNK = 16384
NCH = W // CHUNK
ROWS, COLS = 512, 8192
RBLK = 64
NLANES = 16
UNROLL = 8


GBATCH = 2
GSTEPS = 16 // GBATCH
GBLK = GBATCH * 512 * 512


def _stats_body(l_ref, t_ref, e_ref, s16_ref, max_ref, p_ref, msc_ref, psc_ref):
    i = pl.program_id(0)
    l = l_ref[...].reshape(GBATCH * 512, 512)
    t = t_ref[...].reshape(GBATCH * 512, 512)
    tf = t.astype(jnp.float32)
    e = 1.0 - l * (2.0 * tf - 1.0)
    ebits = lax.bitcast_convert_type(e, jnp.int32)
    epacked = lax.bitcast_convert_type(
        jnp.bitwise_or(jnp.bitwise_and(ebits, jnp.int32(-2)), t), jnp.float32)
    e_ref[...] = epacked.reshape(GBLK)
    bmax = jnp.max(epacked)
    bsum = jnp.sum(tf)

    @pl.when(i == 0)
    def _():
        msc_ref[0] = bmax
        psc_ref[0] = bsum

    @pl.when(i > 0)
    def _():
        msc_ref[0] = jnp.maximum(msc_ref[0], bmax)
        psc_ref[0] = psc_ref[0] + bsum

    @pl.when(i == GSTEPS - 1)
    def _():
        max_ref[0, 0] = msc_ref[0]
        p_ref[0, 0] = psc_ref[0]
        s16_ref[...] = jnp.full((NLANES,), msc_ref[0], jnp.float32)


def _stats(logits4d, targets4d):
    return pl.pallas_call(
        _stats_body,
        grid=(GSTEPS,),
        in_specs=[
            pl.BlockSpec((GBATCH, 1, 512, 512), lambda i: (i, 0, 0, 0)),
            pl.BlockSpec((GBATCH, 1, 512, 512), lambda i: (i, 0, 0, 0)),
        ],
        out_specs=[
            pl.BlockSpec((GBLK,), lambda i: (i,)),
            pl.BlockSpec((NLANES,), lambda i: (0,)),
            pl.BlockSpec(memory_space=pltpu.SMEM),
            pl.BlockSpec(memory_space=pltpu.SMEM),
        ],
        out_shape=[
            jax.ShapeDtypeStruct((M,), jnp.float32),
            jax.ShapeDtypeStruct((NLANES,), jnp.float32),
            jax.ShapeDtypeStruct((1, 1), jnp.float32),
            jax.ShapeDtypeStruct((1, 1), jnp.float32),
        ],
        scratch_shapes=[
            pltpu.SMEM((1,), jnp.float32),
            pltpu.SMEM((1,), jnp.float32),
        ],
    )(logits4d, targets4d)


_mesh = plsc.VectorSubcoreMesh(core_axis_name="c", subcore_axis_name="s")


@functools.partial(
    pl.kernel,
    mesh=_mesh,
    compiler_params=pltpu.CompilerParams(needs_layout_passes=False),
    out_type=jax.ShapeDtypeStruct((NW, 2 * B), jnp.float32),
    scratch_types=[
        pltpu.VMEM((CHUNK,), jnp.float32),
        pltpu.VMEM((CHUNK,), jnp.float32),
        pltpu.VMEM((NLANES,), jnp.float32),
        pltpu.VMEM((NLANES * (2 * B + NLANES),), jnp.float32),
        pltpu.VMEM((2 * B,), jnp.float32),
        pltpu.SemaphoreType.DMA,
        pltpu.SemaphoreType.DMA,
    ],
)
def _sc_hist(e_hbm, s_hbm, out_hbm, bufa, bufb, sbuf, hist, rhist, sema, semb):
    wid = lax.axis_index("s") * 2 + lax.axis_index("c")
    base = wid * W

    pltpu.sync_copy(s_hbm, sbuf)
    tmax = jnp.maximum(sbuf[...], 1e-30)
    invd = jnp.float32(B) / tmax
    lane = lax.broadcasted_iota(jnp.int32, (NLANES,), 0)
    lane_base = lane * (2 * B + NLANES) + lane
    ones = jnp.full((NLANES,), 1.0, jnp.float32)

    def zbody(i, _):
        hist[pl.ds(i * NLANES, NLANES)] = jnp.zeros((NLANES,), jnp.float32)
        return ()

    lax.fori_loop(0, NLANES * (2 * B + NLANES) // NLANES, zbody, ())

    def process(buf):
        @plsc.parallel_loop(0, CHUNK // NLANES, unroll=UNROLL)
        def _(i):
            v = buf[pl.ds(i * NLANES, NLANES)]
            ti = jnp.bitwise_and(plsc.bitcast(v, jnp.int32), 1)
            bi = jnp.minimum((v * invd).astype(jnp.int32), B - 1)
            idx = lane_base + ti * B + bi
            plsc.addupdate_scatter(hist, [idx], ones, mask=v > 0.0)

    def start(chunk, buf, sem):
        pltpu.async_copy(e_hbm.at[pl.ds(base + chunk * CHUNK, CHUNK)], buf, sem)

    def wait(chunk, buf, sem):
        pltpu.make_async_copy(
            e_hbm.at[pl.ds(base + chunk * CHUNK, CHUNK)], buf, sem).wait()

    start(0, bufa, sema)

    def chunk_body(h, _):
        ca = 2 * h
        start(ca + 1, bufb, semb)
        wait(ca, bufa, sema)
        process(bufa)
        start(jnp.minimum(ca + 2, NCH - 1), bufa, sema)
        wait(ca + 1, bufb, semb)
        process(bufb)
        return ()

    lax.fori_loop(0, NCH // 2, chunk_body, ())
    wait(NCH - 1, bufa, sema)

    def rbody(j, _):
        acc = jnp.zeros((NLANES,), jnp.float32)
        for ln in range(NLANES):
            acc = acc + hist[pl.ds(ln * (2 * B + NLANES) + ln + j * NLANES,
                                   NLANES)]
        rhist[pl.ds(j * NLANES, NLANES)] = acc
        return ()

    lax.fori_loop(0, 2 * B // NLANES, rbody, ())
    pltpu.sync_copy(rhist, out_hbm.at[wid])


def _finish_body(h_ref, tmax_ref, p_ref, out_ref):
    h = h_ref[...]
    hp = jnp.sum(h[:, :B], axis=0, keepdims=True)
    hn = jnp.sum(h[:, B:], axis=0, keepdims=True)
    r = lax.broadcasted_iota(jnp.int32, (B, 1), 0)
    c = lax.broadcasted_iota(jnp.int32, (1, B), 1)
    upper = (r >= c).astype(jnp.float32)
    sp = jnp.dot(hp, upper, precision=lax.Precision.HIGHEST,
                 preferred_element_type=jnp.float32)
    sn = jnp.dot(hn, upper, precision=lax.Precision.HIGHEST,
                 preferred_element_type=jnp.float32)
    p_tot = p_ref[0, 0]
    tmax = jnp.maximum(tmax_ref[0, 0], 1e-30)
    jac = 1.0 - (p_tot - sp) / jnp.maximum(p_tot + sn, 1.0)
    delta = tmax / jnp.float32(B)
    total = delta * (jnp.sum(jac) - 0.5 * jac[0, 0])
    out_ref[0, 0] = jnp.where(p_tot > 0.0, total, 0.0)


def _finish(hists, tmax, p_tot):
    return pl.pallas_call(
        _finish_body,
        in_specs=[
            pl.BlockSpec((NW, 2 * B), lambda: (0, 0)),
            pl.BlockSpec(memory_space=pltpu.SMEM),
            pl.BlockSpec(memory_space=pltpu.SMEM),
        ],
        out_specs=pl.BlockSpec(memory_space=pltpu.SMEM),
        out_shape=jax.ShapeDtypeStruct((1, 1), jnp.float32),
    )(hists, tmax, p_tot)


def kernel(inputs, targets):
    epacked, stats16, tmax, p_tot = _stats(inputs, targets)
    hists = _sc_hist(epacked, stats16)
    loss = _finish(hists, tmax, p_tot)
    return loss.reshape(())


# speedup vs baseline: 96.9629x; 1.1050x over previous
import functools

import jax
import jax.numpy as jnp
from jax import lax
from jax.experimental import pallas as pl
from jax.experimental.pallas import tpu as pltpu
from jax.experimental.pallas import tpu_sc as plsc

M = 16 * 512 * 512
B = 2048
NW = 32
W = M // NW
CHUNK = 16384
NCH = W // CHUNK
ROWS, COLS = 512, 8192
RBLK = 64
NLANES = 16
UNROLL = 16


GBATCH = 2
GSTEPS = 16 // GBATCH
GBLK = GBATCH * 512 * 512


def _stats_body(l_ref, t_ref, e_ref, s16_ref, max_ref, p_ref, msc_ref, psc_ref):
    i = pl.program_id(0)
    l = l_ref[...].reshape(GBATCH * 512, 512)
    t = t_ref[...].reshape(GBATCH * 512, 512)
    tf = t.astype(jnp.float32)
    e = 1.0 - l * (2.0 * tf - 1.0)
    ebits = lax.bitcast_convert_type(e, jnp.int32)
    epacked = lax.bitcast_convert_type(
        jnp.bitwise_or(jnp.bitwise_and(ebits, jnp.int32(-2)), t), jnp.float32)
    e_ref[...] = epacked.reshape(GBLK)
    bmax = jnp.max(epacked)
    bsum = jnp.sum(tf)

    @pl.when(i == 0)
    def _():
        msc_ref[0] = bmax
        psc_ref[0] = bsum

    @pl.when(i > 0)
    def _():
        msc_ref[0] = jnp.maximum(msc_ref[0], bmax)
        psc_ref[0] = psc_ref[0] + bsum

    @pl.when(i == GSTEPS - 1)
    def _():
        max_ref[0, 0] = msc_ref[0]
        p_ref[0, 0] = psc_ref[0]
        s16_ref[...] = jnp.full((NLANES,), msc_ref[0], jnp.float32)


def _stats(logits4d, targets4d):
    return pl.pallas_call(
        _stats_body,
        grid=(GSTEPS,),
        in_specs=[
            pl.BlockSpec((GBATCH, 1, 512, 512), lambda i: (i, 0, 0, 0)),
            pl.BlockSpec((GBATCH, 1, 512, 512), lambda i: (i, 0, 0, 0)),
        ],
        out_specs=[
            pl.BlockSpec((GBLK,), lambda i: (i,)),
            pl.BlockSpec((NLANES,), lambda i: (0,)),
            pl.BlockSpec(memory_space=pltpu.SMEM),
            pl.BlockSpec(memory_space=pltpu.SMEM),
        ],
        out_shape=[
            jax.ShapeDtypeStruct((M,), jnp.float32),
            jax.ShapeDtypeStruct((NLANES,), jnp.float32),
            jax.ShapeDtypeStruct((1, 1), jnp.float32),
            jax.ShapeDtypeStruct((1, 1), jnp.float32),
        ],
        scratch_shapes=[
            pltpu.SMEM((1,), jnp.float32),
            pltpu.SMEM((1,), jnp.float32),
        ],
    )(logits4d, targets4d)


_mesh = plsc.VectorSubcoreMesh(core_axis_name="c", subcore_axis_name="s")


@functools.partial(
    pl.kernel,
    mesh=_mesh,
    compiler_params=pltpu.CompilerParams(needs_layout_passes=False),
    out_type=jax.ShapeDtypeStruct((NW, 2 * B), jnp.float32),
    scratch_types=[
        pltpu.VMEM((CHUNK,), jnp.float32),
        pltpu.VMEM((CHUNK,), jnp.float32),
        pltpu.VMEM((NLANES,), jnp.float32),
        pltpu.VMEM((NLANES * (2 * B + NLANES),), jnp.float32),
        pltpu.VMEM((2 * B,), jnp.float32),
        pltpu.SemaphoreType.DMA,
        pltpu.SemaphoreType.DMA,
    ],
)
def _sc_hist(e_hbm, s_hbm, out_hbm, bufa, bufb, sbuf, hist, rhist, sema, semb):
    wid = lax.axis_index("s") * 2 + lax.axis_index("c")
    base = wid * W

    pltpu.sync_copy(s_hbm, sbuf)
    tmax = jnp.maximum(sbuf[...], 1e-30)
    invd = jnp.float32(B) / tmax
    lane = lax.broadcasted_iota(jnp.int32, (NLANES,), 0)
    lane_base = lane * (2 * B + NLANES) + lane
    ones = jnp.full((NLANES,), 1.0, jnp.float32)

    def zbody(i, _):
        hist[pl.ds(i * NLANES, NLANES)] = jnp.zeros((NLANES,), jnp.float32)
        return ()

    lax.fori_loop(0, NLANES * (2 * B + NLANES) // NLANES, zbody, ())

    def process(buf):
        @plsc.parallel_loop(0, CHUNK // NLANES, unroll=UNROLL)
        def _(i):
            v = buf[pl.ds(i * NLANES, NLANES)]
            ti = jnp.bitwise_and(plsc.bitcast(v, jnp.int32), 1)
            bi = (v * invd).astype(jnp.int32)
            idx = lane_base + ti * B + bi
            plsc.addupdate_scatter(hist, [idx], ones, mask=v > 0.0)

    def start(chunk, buf, sem):
        pltpu.async_copy(e_hbm.at[pl.ds(base + chunk * CHUNK, CHUNK)], buf, sem)

    def wait(chunk, buf, sem):
        pltpu.make_async_copy(
            e_hbm.at[pl.ds(base + chunk * CHUNK, CHUNK)], buf, sem).wait()

    start(0, bufa, sema)

    def chunk_body(h, _):
        ca = 2 * h
        start(ca + 1, bufb, semb)
        wait(ca, bufa, sema)
        process(bufa)
        start(jnp.minimum(ca + 2, NCH - 1), bufa, sema)
        wait(ca + 1, bufb, semb)
        process(bufb)
        return ()

    lax.fori_loop(0, NCH // 2, chunk_body, ())
    wait(NCH - 1, bufa, sema)

    def rbody(j, _):
        acc = jnp.zeros((NLANES,), jnp.float32)
        for ln in range(NLANES):
            acc = acc + hist[pl.ds(ln * (2 * B + NLANES) + ln + j * NLANES,
                                   NLANES)]
        rhist[pl.ds(j * NLANES, NLANES)] = acc
        return ()

    lax.fori_loop(0, 2 * B // NLANES, rbody, ())
    pltpu.sync_copy(rhist, out_hbm.at[wid])


BR, BC = 16, B // 16


def _suffix(h2, uc, ur):
    within = jnp.dot(h2, uc, precision=lax.Precision.HIGHEST,
                     preferred_element_type=jnp.float32)
    tot = jnp.sum(h2, axis=1, keepdims=True)
    later = jnp.dot(ur, tot, precision=lax.Precision.HIGHEST,
                    preferred_element_type=jnp.float32)
    return within + later


def _finish_body(h_ref, tmax_ref, p_ref, out_ref):
    h = h_ref[...]
    hp = jnp.sum(h[:, :B], axis=0).reshape(BR, BC)
    hn = jnp.sum(h[:, B:], axis=0).reshape(BR, BC)
    rj = lax.broadcasted_iota(jnp.int32, (BC, 1), 0)
    ck = lax.broadcasted_iota(jnp.int32, (1, BC), 1)
    uc = (rj >= ck).astype(jnp.float32)
    rr = lax.broadcasted_iota(jnp.int32, (BR, 1), 0)
    cr = lax.broadcasted_iota(jnp.int32, (1, BR), 1)
    ur = (cr > rr).astype(jnp.float32)
    sp = _suffix(hp, uc, ur)
    sn = _suffix(hn, uc, ur)
    p_tot = p_ref[0, 0]
    tmax = jnp.maximum(tmax_ref[0, 0], 1e-30)
    jac = 1.0 - (p_tot - sp) / jnp.maximum(p_tot + sn, 1.0)
    delta = tmax / jnp.float32(B)
    total = delta * (jnp.sum(jac) - 0.5 * jac[0, 0])
    out_ref[0, 0] = jnp.where(p_tot > 0.0, total, 0.0)


def _finish(hists, tmax, p_tot):
    return pl.pallas_call(
        _finish_body,
        in_specs=[
            pl.BlockSpec((NW, 2 * B), lambda: (0, 0)),
            pl.BlockSpec(memory_space=pltpu.SMEM),
            pl.BlockSpec(memory_space=pltpu.SMEM),
        ],
        out_specs=pl.BlockSpec(memory_space=pltpu.SMEM),
        out_shape=jax.ShapeDtypeStruct((1, 1), jnp.float32),
    )(hists, tmax, p_tot)


def kernel(inputs, targets):
    epacked, stats16, tmax, p_tot = _stats(inputs, targets)
    hists = _sc_hist(epacked, stats16)
    loss = _finish(hists, tmax, p_tot)
    return loss.reshape(())


# speedup vs baseline: 107.8150x; 1.1119x over previous
import functools

import jax
import jax.numpy as jnp
from jax import lax
from jax.experimental import pallas as pl
from jax.experimental.pallas import tpu as pltpu
from jax.experimental.pallas import tpu_sc as plsc

M = 16 * 512 * 512
B = 1024
NW = 32
W = M // NW
CHUNK = 16384
NCH = W // CHUNK
ROWS, COLS = 512, 8192
RBLK = 64
NLANES = 16
UNROLL = 16


GBATCH = 2
GSTEPS = 16 // GBATCH
GBLK = GBATCH * 512 * 512


def _stats_body(l_ref, t_ref, e_ref, s16_ref, max_ref, p_ref, msc_ref, psc_ref):
    i = pl.program_id(0)
    l = l_ref[...].reshape(GBATCH * 512, 512)
    t = t_ref[...].reshape(GBATCH * 512, 512)
    tf = t.astype(jnp.float32)
    e = 1.0 - l * (2.0 * tf - 1.0)
    ebits = lax.bitcast_convert_type(e, jnp.int32)
    epacked = lax.bitcast_convert_type(
        jnp.bitwise_or(jnp.bitwise_and(ebits, jnp.int32(-2)), t), jnp.float32)
    e_ref[...] = epacked.reshape(GBLK)
    bmax = jnp.max(epacked)
    bsum = jnp.sum(tf)

    @pl.when(i == 0)
    def _():
        msc_ref[0] = bmax
        psc_ref[0] = bsum

    @pl.when(i > 0)
    def _():
        msc_ref[0] = jnp.maximum(msc_ref[0], bmax)
        psc_ref[0] = psc_ref[0] + bsum

    @pl.when(i == GSTEPS - 1)
    def _():
        max_ref[0, 0] = msc_ref[0]
        p_ref[0, 0] = psc_ref[0]
        s16_ref[...] = jnp.full((NLANES,), msc_ref[0], jnp.float32)


def _stats(logits4d, targets4d):
    return pl.pallas_call(
        _stats_body,
        grid=(GSTEPS,),
        in_specs=[
            pl.BlockSpec((GBATCH, 1, 512, 512), lambda i: (i, 0, 0, 0)),
            pl.BlockSpec((GBATCH, 1, 512, 512), lambda i: (i, 0, 0, 0)),
        ],
        out_specs=[
            pl.BlockSpec((GBLK,), lambda i: (i,)),
            pl.BlockSpec((NLANES,), lambda i: (0,)),
            pl.BlockSpec(memory_space=pltpu.SMEM),
            pl.BlockSpec(memory_space=pltpu.SMEM),
        ],
        out_shape=[
            jax.ShapeDtypeStruct((M,), jnp.float32),
            jax.ShapeDtypeStruct((NLANES,), jnp.float32),
            jax.ShapeDtypeStruct((1, 1), jnp.float32),
            jax.ShapeDtypeStruct((1, 1), jnp.float32),
        ],
        scratch_shapes=[
            pltpu.SMEM((1,), jnp.float32),
            pltpu.SMEM((1,), jnp.float32),
        ],
    )(logits4d, targets4d)


_mesh = plsc.VectorSubcoreMesh(core_axis_name="c", subcore_axis_name="s")


@functools.partial(
    pl.kernel,
    mesh=_mesh,
    compiler_params=pltpu.CompilerParams(needs_layout_passes=False),
    out_type=jax.ShapeDtypeStruct((NW, 2 * B), jnp.float32),
    scratch_types=[
        pltpu.VMEM((CHUNK,), jnp.float32),
        pltpu.VMEM((CHUNK,), jnp.float32),
        pltpu.VMEM((NLANES,), jnp.float32),
        pltpu.VMEM((NLANES * (2 * B + NLANES),), jnp.float32),
        pltpu.VMEM((NLANES * (2 * B + NLANES),), jnp.float32),
        pltpu.VMEM((2 * B,), jnp.float32),
        pltpu.SemaphoreType.DMA,
        pltpu.SemaphoreType.DMA,
    ],
)
def _sc_hist(e_hbm, s_hbm, out_hbm, bufa, bufb, sbuf, hist, hist2, rhist,
             sema, semb):
    wid = lax.axis_index("s") * 2 + lax.axis_index("c")
    base = wid * W

    pltpu.sync_copy(s_hbm, sbuf)
    tmax = jnp.maximum(sbuf[...], 1e-30)
    invd = jnp.float32(B) / tmax
    lane = lax.broadcasted_iota(jnp.int32, (NLANES,), 0)
    lane_base = lane * (2 * B + NLANES) + lane
    ones = jnp.full((NLANES,), 1.0, jnp.float32)

    def zbody(i, _):
        hist[pl.ds(i * NLANES, NLANES)] = jnp.zeros((NLANES,), jnp.float32)
        hist2[pl.ds(i * NLANES, NLANES)] = jnp.zeros((NLANES,), jnp.float32)
        return ()

    lax.fori_loop(0, NLANES * (2 * B + NLANES) // NLANES, zbody, ())

    def process(buf):
        @plsc.parallel_loop(0, CHUNK // NLANES, step=2, unroll=UNROLL // 2)
        def _(i):
            for half, hh in ((0, hist), (1, hist2)):
                v = buf[pl.ds((i + half) * NLANES, NLANES)]
                ti = jnp.bitwise_and(plsc.bitcast(v, jnp.int32), 1)
                bi = (v * invd).astype(jnp.int32)
                idx = lane_base + ti * B + bi
                plsc.addupdate_scatter(hh, [idx], ones, mask=v > 0.0)

    def start(chunk, buf, sem):
        pltpu.async_copy(e_hbm.at[pl.ds(base + chunk * CHUNK, CHUNK)], buf, sem)

    def wait(chunk, buf, sem):
        pltpu.make_async_copy(
            e_hbm.at[pl.ds(base + chunk * CHUNK, CHUNK)], buf, sem).wait()

    start(0, bufa, sema)

    def chunk_body(h, _):
        ca = 2 * h
        start(ca + 1, bufb, semb)
        wait(ca, bufa, sema)
        process(bufa)
        start(jnp.minimum(ca + 2, NCH - 1), bufa, sema)
        wait(ca + 1, bufb, semb)
        process(bufb)
        return ()

    lax.fori_loop(0, NCH // 2, chunk_body, ())
    wait(NCH - 1, bufa, sema)

    def rbody(j, _):
        acc = jnp.zeros((NLANES,), jnp.float32)
        for ln in range(NLANES):
            off = ln * (2 * B + NLANES) + ln + j * NLANES
            acc = acc + hist[pl.ds(off, NLANES)] + hist2[pl.ds(off, NLANES)]
        rhist[pl.ds(j * NLANES, NLANES)] = acc
        return ()

    lax.fori_loop(0, 2 * B // NLANES, rbody, ())
    pltpu.sync_copy(rhist, out_hbm.at[wid])


BR, BC = 8, B // 8


def _suffix(h2, uc, ur):
    within = jnp.dot(h2, uc, precision=lax.Precision.HIGHEST,
                     preferred_element_type=jnp.float32)
    tot = jnp.sum(h2, axis=1, keepdims=True)
    later = jnp.dot(ur, tot, precision=lax.Precision.HIGHEST,
                    preferred_element_type=jnp.float32)
    return within + later


def _finish_body(h_ref, tmax_ref, p_ref, out_ref):
    h = h_ref[...]
    hp = jnp.sum(h[:, :B], axis=0).reshape(BR, BC)
    hn = jnp.sum(h[:, B:], axis=0).reshape(BR, BC)
    rj = lax.broadcasted_iota(jnp.int32, (BC, 1), 0)
    ck = lax.broadcasted_iota(jnp.int32, (1, BC), 1)
    uc = (rj >= ck).astype(jnp.float32)
    rr = lax.broadcasted_iota(jnp.int32, (BR, 1), 0)
    cr = lax.broadcasted_iota(jnp.int32, (1, BR), 1)
    ur = (cr > rr).astype(jnp.float32)
    sp = _suffix(hp, uc, ur)
    sn = _suffix(hn, uc, ur)
    p_tot = p_ref[0, 0]
    tmax = jnp.maximum(tmax_ref[0, 0], 1e-30)
    jac = 1.0 - (p_tot - sp) / jnp.maximum(p_tot + sn, 1.0)
    delta = tmax / jnp.float32(B)
    total = delta * (jnp.sum(jac) - 0.5 * jac[0, 0])
    out_ref[0, 0] = jnp.where(p_tot > 0.0, total, 0.0)


def _finish(hists, tmax, p_tot):
    return pl.pallas_call(
        _finish_body,
        in_specs=[
            pl.BlockSpec((NW, 2 * B), lambda: (0, 0)),
            pl.BlockSpec(memory_space=pltpu.SMEM),
            pl.BlockSpec(memory_space=pltpu.SMEM),
        ],
        out_specs=pl.BlockSpec(memory_space=pltpu.SMEM),
        out_shape=jax.ShapeDtypeStruct((1, 1), jnp.float32),
    )(hists, tmax, p_tot)


def kernel(inputs, targets):
    epacked, stats16, tmax, p_tot = _stats(inputs, targets)
    hists = _sc_hist(epacked, stats16)
    loss = _finish(hists, tmax, p_tot)
    return loss.reshape(())
